# Initial kernel scaffold; baseline (speedup 1.0000x reference)
import jax
import jax.numpy as jnp
from jax.experimental import pallas as pl


def kernel(X, batch, ct_size, n_ct, W1, b1, Wc, bc, Wct, bct, Wo, bo):
    raise NotImplementedError("write your pallas kernel here")


# speedup vs baseline: 10.7371x; 10.7371x over previous
import functools

import jax
import jax.numpy as jnp
from jax import lax
from jax.experimental import pallas as pl
from jax.experimental.pallas import tpu as pltpu
from jax.experimental.pallas import tpu_sc as plsc

N = 320000
N_IN = 128
N_HID = 32
SEGS = 8000
N_CT = 16
NBAGS = SEGS // N_CT

GW = 48
BLK_A = 4000

NW = 32
RW = N // NW
SUB = 125
NSUB = 16
BIG = SUB * NSUB
NBIG = RW // BIG
BROWS = N // SUB
ZROWS = SEGS // 8


def _stage_a(x_ref, w1_ref, b1_ref, wc_ref, bc_ref, g_ref):
    x = x_ref[...]
    h = jnp.maximum(
        jnp.dot(x, w1_ref[...], preferred_element_type=jnp.float32) + b1_ref[...],
        0.0,
    )
    s = jnp.sum(h * wc_ref[...], axis=1, keepdims=True) + bc_ref[...]
    e = jnp.exp(s)
    g_ref[...] = jnp.concatenate(
        [h * e, e, jnp.zeros((BLK_A, GW - N_HID - 1), jnp.float32)], axis=1
    )


def _stage_b(g_hbm, b_hbm, out_hbm, gbuf, ibuf, uacc):
    cid = lax.axis_index("c")
    sid = lax.axis_index("s")
    wid = sid * 2 + cid

    z = jnp.zeros((16,), jnp.float32)

    def zrow(i, carry):
        gbuf[i, pl.ds(0, 16)] = z
        gbuf[i, pl.ds(16, 16)] = z
        gbuf[i, pl.ds(32, 16)] = z
        return carry

    @pl.when(sid < 8)
    def _zero():
        lax.fori_loop(0, ZROWS, zrow, 0)
        pltpu.sync_copy(
            gbuf.at[pl.ds(0, ZROWS)], uacc.at[pl.ds(sid * ZROWS, ZROWS)]
        )

    plsc.subcore_barrier()

    rowbase = wid * RW

    def big_iter(bi, carry):
        pltpu.sync_copy(g_hbm.at[pl.ds(rowbase + bi * BIG, BIG)], gbuf)
        pltpu.sync_copy(b_hbm.at[pl.ds(wid * (RW // SUB) + bi * NSUB, NSUB)], ibuf)

        def sub_iter(j, c2):
            pltpu.sync_copy(
                gbuf.at[pl.ds(j * SUB, SUB)], uacc.at[ibuf.at[j]], add=True
            )
            return c2

        lax.fori_loop(0, NSUB, sub_iter, 0)
        return carry

    lax.fori_loop(0, NBIG, big_iter, 0)
    plsc.subcore_barrier()

    @pl.when(sid < 8)
    def _flush():
        pltpu.sync_copy(
            uacc.at[pl.ds(sid * ZROWS, ZROWS)],
            out_hbm.at[cid, pl.ds(sid * ZROWS, ZROWS)],
        )


def _stage_c(u_ref, wct_ref, bct_ref, wo_ref, bo_ref, out_ref):
    u = u_ref[...]
    uc = u[0] + u[1]
    pooled = uc[:, :, :N_HID] / (uc[:, :, N_HID:N_HID + 1] + 1e-16)
    t = jnp.sum(pooled * wct_ref[...], axis=-1, keepdims=True) + bct_ref[...]
    m = jnp.max(t, axis=1, keepdims=True)
    ee = jnp.exp(t - m)
    dd = jnp.sum(ee, axis=1, keepdims=True)
    xs = jnp.sum(pooled * (ee / dd), axis=1)
    out_ref[...] = jnp.sum(xs * wo_ref[...], axis=-1, keepdims=True) + bo_ref[...]


def kernel(X, batch, ct_size, n_ct, W1, b1, Wc, bc, Wct, bct, Wo, bo):
    g = pl.pallas_call(
        _stage_a,
        grid=(N // BLK_A,),
        in_specs=[
            pl.BlockSpec((BLK_A, N_IN), lambda i: (i, 0)),
            pl.BlockSpec((N_IN, N_HID), lambda i: (0, 0)),
            pl.BlockSpec((1, N_HID), lambda i: (0, 0)),
            pl.BlockSpec((1, N_HID), lambda i: (0, 0)),
            pl.BlockSpec((1, 1), lambda i: (0, 0)),
        ],
        out_specs=pl.BlockSpec((BLK_A, GW), lambda i: (i, 0)),
        out_shape=jax.ShapeDtypeStruct((N, GW), jnp.float32),
    )(
        X,
        W1,
        b1.reshape(1, N_HID),
        Wc[:, 0].reshape(1, N_HID),
        bc.reshape(1, 1),
    )

    mesh = plsc.VectorSubcoreMesh(core_axis_name="c", subcore_axis_name="s")
    u2 = pl.kernel(
        _stage_b,
        mesh=mesh,
        compiler_params=pltpu.CompilerParams(use_tc_tiling_on_sc=False),
        out_type=jax.ShapeDtypeStruct((2, SEGS, GW), jnp.float32),
        scratch_types=[
            pltpu.VMEM((BIG, GW), jnp.float32),
            pltpu.VMEM((NSUB, SUB), jnp.int32),
            pltpu.VMEM_SHARED((SEGS, GW), jnp.float32),
        ],
    )(g, batch.reshape(BROWS, SUB))

    out = pl.pallas_call(
        _stage_c,
        out_shape=jax.ShapeDtypeStruct((NBAGS, 1), jnp.float32),
    )(
        u2.reshape(2, NBAGS, N_CT, GW),
        Wct[:, 0].reshape(1, 1, N_HID),
        bct.reshape(1, 1, 1),
        Wo[:, 0].reshape(1, N_HID),
        bo.reshape(1, 1),
    )
    return out


# speedup vs baseline: 11.0372x; 1.0280x over previous
import functools

import jax
import jax.numpy as jnp
from jax import lax
from jax.experimental import pallas as pl
from jax.experimental.pallas import tpu as pltpu
from jax.experimental.pallas import tpu_sc as plsc

N = 320000
N_IN = 128
N_HID = 32
SEGS = 8000
N_CT = 16
NBAGS = SEGS // N_CT

GW = 48
BLK_A = 4000

NW = 32
RW = N // NW
SUB = 125
NSUB = 8
BIG = SUB * NSUB
NBIG = RW // BIG
BROWS = N // SUB
ZROWS = SEGS // 8


def _stage_a(x_ref, w1_ref, b1_ref, wc_ref, bc_ref, g_ref):
    x = x_ref[...]
    h = jnp.maximum(
        jnp.dot(x, w1_ref[...], preferred_element_type=jnp.float32) + b1_ref[...],
        0.0,
    )
    s = jnp.sum(h * wc_ref[...], axis=1, keepdims=True) + bc_ref[...]
    e = jnp.exp(s)
    g_ref[...] = jnp.concatenate(
        [h * e, e, jnp.zeros((BLK_A, GW - N_HID - 1), jnp.float32)], axis=1
    )


def _stage_b(g_hbm, b_hbm, out_hbm, gbuf, ibuf, uacc, lsem, ssem):
    cid = lax.axis_index("c")
    sid = lax.axis_index("s")
    wid = sid * 2 + cid

    z = jnp.zeros((16,), jnp.float32)

    def zrow(i, carry):
        gbuf[0, i, pl.ds(0, 16)] = z
        gbuf[0, i, pl.ds(16, 16)] = z
        gbuf[0, i, pl.ds(32, 16)] = z
        return carry

    @pl.when(sid < 8)
    def _zero():
        lax.fori_loop(0, ZROWS, zrow, 0)
        pltpu.sync_copy(gbuf.at[0], uacc.at[pl.ds(sid * ZROWS, ZROWS)])

    plsc.subcore_barrier()

    rowbase = wid * RW
    ibase = wid * (RW // SUB)

    def start_load(bi, b):
        hg = pltpu.async_copy(
            g_hbm.at[pl.ds(rowbase + bi * BIG, BIG)], gbuf.at[b], lsem
        )
        hi = pltpu.async_copy(
            b_hbm.at[pl.ds(ibase + bi * NSUB, NSUB)], ibuf.at[b], lsem
        )
        return hg, hi

    pend = start_load(0, 0)
    for bi in range(NBIG):
        b = bi % 2
        hg, hi = pend
        if bi + 1 < NBIG:
            pend = start_load(bi + 1, (bi + 1) % 2)
        hg.wait()
        hi.wait()
        scs = [
            pltpu.async_copy(
                gbuf.at[b, pl.ds(j * SUB, SUB)],
                uacc.at[ibuf.at[b, j]],
                ssem,
                add=True,
            )
            for j in range(NSUB)
        ]
        for h in scs:
            h.wait()

    plsc.subcore_barrier()

    @pl.when(sid < 8)
    def _flush():
        pltpu.sync_copy(
            uacc.at[pl.ds(sid * ZROWS, ZROWS)],
            out_hbm.at[cid, pl.ds(sid * ZROWS, ZROWS)],
        )


def _stage_c(u_ref, wct_ref, bct_ref, wo_ref, bo_ref, out_ref):
    u = u_ref[...]
    uc = u[0] + u[1]
    pooled = uc[:, :, :N_HID] / (uc[:, :, N_HID:N_HID + 1] + 1e-16)
    t = jnp.sum(pooled * wct_ref[...], axis=-1, keepdims=True) + bct_ref[...]
    m = jnp.max(t, axis=1, keepdims=True)
    ee = jnp.exp(t - m)
    dd = jnp.sum(ee, axis=1, keepdims=True)
    xs = jnp.sum(pooled * (ee / dd), axis=1)
    out_ref[...] = jnp.sum(xs * wo_ref[...], axis=-1, keepdims=True) + bo_ref[...]


def kernel(X, batch, ct_size, n_ct, W1, b1, Wc, bc, Wct, bct, Wo, bo):
    g = pl.pallas_call(
        _stage_a,
        grid=(N // BLK_A,),
        in_specs=[
            pl.BlockSpec((BLK_A, N_IN), lambda i: (i, 0)),
            pl.BlockSpec((N_IN, N_HID), lambda i: (0, 0)),
            pl.BlockSpec((1, N_HID), lambda i: (0, 0)),
            pl.BlockSpec((1, N_HID), lambda i: (0, 0)),
            pl.BlockSpec((1, 1), lambda i: (0, 0)),
        ],
        out_specs=pl.BlockSpec((BLK_A, GW), lambda i: (i, 0)),
        out_shape=jax.ShapeDtypeStruct((N, GW), jnp.float32),
    )(
        X,
        W1,
        b1.reshape(1, N_HID),
        Wc[:, 0].reshape(1, N_HID),
        bc.reshape(1, 1),
    )

    mesh = plsc.VectorSubcoreMesh(core_axis_name="c", subcore_axis_name="s")
    u2 = pl.kernel(
        _stage_b,
        mesh=mesh,
        compiler_params=pltpu.CompilerParams(use_tc_tiling_on_sc=False),
        out_type=jax.ShapeDtypeStruct((2, SEGS, GW), jnp.float32),
        scratch_types=[
            pltpu.VMEM((2, BIG, GW), jnp.float32),
            pltpu.VMEM((2, NSUB, SUB), jnp.int32),
            pltpu.VMEM_SHARED((SEGS, GW), jnp.float32),
            pltpu.SemaphoreType.DMA,
            pltpu.SemaphoreType.DMA,
        ],
    )(g, batch.reshape(BROWS, SUB))

    out = pl.pallas_call(
        _stage_c,
        out_shape=jax.ShapeDtypeStruct((NBAGS, 1), jnp.float32),
    )(
        u2.reshape(2, NBAGS, N_CT, GW),
        Wct[:, 0].reshape(1, 1, N_HID),
        bct.reshape(1, 1, 1),
        Wo[:, 0].reshape(1, N_HID),
        bo.reshape(1, 1),
    )
    return out
